# Initial kernel scaffold; baseline (speedup 1.0000x reference)
import jax
import jax.numpy as jnp
from jax.experimental import pallas as pl


def kernel(x, edge_index, W1, b1, W2, b2):
    raise NotImplementedError("write your pallas kernel here")


# speedup vs baseline: 87.2300x; 87.2300x over previous
import functools

import jax
import jax.numpy as jnp
from jax import lax
from jax.experimental import pallas as pl
from jax.experimental.pallas import tpu as pltpu
from jax.experimental.pallas import tpu_sc as plsc

N = 10000
NP = 10240
E = 320000
D = 16
EBUF = 10112
NPAIR = 39

_mesh = plsc.VectorSubcoreMesh(core_axis_name="c", subcore_axis_name="s")
_f32 = jnp.float32
_params = pltpu.CompilerParams(use_tc_tiling_on_sc=False,
                               needs_layout_passes=False)


def _worker_slice(w):
    off = jnp.where(w < 4, w * EBUF, 4 * EBUF + (w - 4) * 9984)
    nch = jnp.where(w < 4, 79, 78)
    return off, nch


@functools.partial(
    pl.kernel,
    out_type=jax.ShapeDtypeStruct((2, NP), _f32),
    mesh=_mesh,
    compiler_params=_params,
    scratch_types=[
        pltpu.VMEM((EBUF,), jnp.int32),
        pltpu.VMEM((128,), _f32),
        pltpu.VMEM((640,), _f32),
        pltpu.VMEM_SHARED((NP,), _f32),
        pltpu.SemaphoreType.DMA,
    ],
)
def _sc_deg(dstE, deg_out, idxbuf, ones, zbuf, deg_sh, sem):
    c = lax.axis_index("c")
    s = lax.axis_index("s")
    w = s * 2 + c
    off, nch = _worker_slice(w)

    zero16 = jnp.zeros((16,), _f32)
    one16 = jnp.ones((16,), _f32)

    def _fill_z(i, _):
        zbuf[pl.ds(i * 16, 16)] = zero16
        return 0

    lax.fori_loop(0, 40, _fill_z, 0)

    def _fill_o(i, _):
        ones[pl.ds(i * 16, 16)] = one16
        return 0

    lax.fori_loop(0, 8, _fill_o, 0)

    pltpu.sync_copy(zbuf, deg_sh.at[pl.ds(s * 640, 640)])
    pltpu.sync_copy(dstE.at[pl.ds(off, EBUF)], idxbuf)
    plsc.subcore_barrier()

    def _chunk(j, _):
        dst_idx = idxbuf.at[pl.ds(j * 128, 128)]
        pltpu.async_copy(ones, deg_sh.at[dst_idx], sem, add=True)

        @pl.when(j >= 4)
        def _():
            pltpu.make_async_copy(ones, deg_sh.at[dst_idx], sem).wait()

        return 0

    lax.fori_loop(0, nch, _chunk, 0)
    for _ in range(4):
        pltpu.make_async_copy(ones, deg_sh.at[idxbuf.at[pl.ds(0, 128)]],
                              sem).wait()
    plsc.subcore_barrier()

    pltpu.sync_copy(deg_sh.at[pl.ds(s * 640, 640)], zbuf)
    pltpu.sync_copy(zbuf, deg_out.at[c, pl.ds(s * 640, 640)])


@functools.partial(
    pl.kernel,
    out_type=jax.ShapeDtypeStruct((2, NP, D), _f32),
    mesh=_mesh,
    compiler_params=_params,
    scratch_types=[
        pltpu.VMEM((EBUF,), jnp.int32),
        pltpu.VMEM((EBUF,), jnp.int32),
        pltpu.VMEM((128, D), _f32),
        pltpu.VMEM((128, D), _f32),
        pltpu.VMEM((128, D), _f32),
        pltpu.VMEM((640, D), _f32),
        pltpu.VMEM_SHARED((NP, D), _f32),
        pltpu.VMEM_SHARED((NP, D), _f32),
        pltpu.SemaphoreType.DMA,
        pltpu.SemaphoreType.DMA,
    ],
)
def _sc_agg_rows(srcE, dstE, g1, p_out, srcbuf, dstbuf, rowsA, rowsB, zbuf,
                 stagebuf, g1_sh, agg_sh, semA, semB):
    c = lax.axis_index("c")
    s = lax.axis_index("s")
    w = s * 2 + c
    off, nch = _worker_slice(w)

    zero16 = jnp.zeros((16,), _f32)

    def _fill_z(i, _):
        zbuf[i, :] = zero16
        return 0

    lax.fori_loop(0, 128, _fill_z, 0)
    for t in range(5):
        pltpu.sync_copy(zbuf, agg_sh.at[pl.ds(s * 640 + t * 128, 128)])

    pltpu.sync_copy(g1.at[pl.ds(s * 640, 640)], stagebuf)
    pltpu.sync_copy(stagebuf, g1_sh.at[pl.ds(s * 640, 640)])

    pltpu.sync_copy(srcE.at[pl.ds(off, EBUF)], srcbuf)
    pltpu.sync_copy(dstE.at[pl.ds(off, EBUF)], dstbuf)
    plsc.subcore_barrier()

    def _src_at(j):
        return srcbuf.at[pl.ds(j * 128, 128)]

    def _dst_at(j):
        return dstbuf.at[pl.ds(j * 128, 128)]

    pltpu.async_copy(g1_sh.at[_src_at(0)], rowsA, semA)

    def _pair(i, carry):
        j0 = 2 * i
        j1 = j0 + 1
        pltpu.make_async_copy(g1_sh.at[_src_at(j0)], rowsA, semA).wait()
        pltpu.async_copy(g1_sh.at[_src_at(j1)], rowsB, semB)
        pltpu.sync_copy(rowsA, agg_sh.at[_dst_at(j0)], add=True)
        pltpu.make_async_copy(g1_sh.at[_src_at(j1)], rowsB, semB).wait()
        pltpu.async_copy(g1_sh.at[_src_at((j1 + 1) % nch)], rowsA, semA)
        pltpu.sync_copy(rowsB, agg_sh.at[_dst_at(j1)], add=True)
        return carry

    lax.fori_loop(0, NPAIR, _pair, 0)
    pltpu.make_async_copy(g1_sh.at[_src_at(0)], rowsA, semA).wait()

    @pl.when(nch == 79)
    def _tail():
        pltpu.sync_copy(rowsA, agg_sh.at[_dst_at(78)], add=True)

    plsc.subcore_barrier()

    for t in range(5):
        pltpu.sync_copy(agg_sh.at[pl.ds(s * 640 + t * 128, 128)], zbuf)
        pltpu.sync_copy(zbuf, p_out.at[c, pl.ds(s * 640 + t * 128, 128)])


@functools.partial(
    pl.kernel,
    out_type=jax.ShapeDtypeStruct((2, NP), _f32),
    mesh=_mesh,
    compiler_params=_params,
    scratch_types=[
        pltpu.VMEM((NP,), _f32),
        pltpu.VMEM((EBUF,), jnp.int32),
        pltpu.VMEM((EBUF,), jnp.int32),
        pltpu.VMEM((128,), _f32),
        pltpu.VMEM((640,), _f32),
        pltpu.VMEM_SHARED((NP,), _f32),
    ],
)
def _sc_agg_scalar(srcE, dstE, g2, q_out, g2buf, srcbuf, dstbuf, msg, zbuf,
                   agg_sh):
    c = lax.axis_index("c")
    s = lax.axis_index("s")
    w = s * 2 + c
    off, nch = _worker_slice(w)

    zero16 = jnp.zeros((16,), _f32)

    def _fill_z(i, _):
        zbuf[pl.ds(i * 16, 16)] = zero16
        return 0

    lax.fori_loop(0, 40, _fill_z, 0)
    pltpu.sync_copy(zbuf, agg_sh.at[pl.ds(s * 640, 640)])
    pltpu.sync_copy(g2, g2buf)
    pltpu.sync_copy(srcE.at[pl.ds(off, EBUF)], srcbuf)
    pltpu.sync_copy(dstE.at[pl.ds(off, EBUF)], dstbuf)
    plsc.subcore_barrier()

    def _chunk(j, _):
        for g in range(8):
            idx16 = srcbuf[pl.ds(j * 128 + g * 16, 16)]
            msg[pl.ds(g * 16, 16)] = plsc.load_gather(g2buf, [idx16])
        pltpu.sync_copy(msg, agg_sh.at[dstbuf.at[pl.ds(j * 128, 128)]],
                        add=True)
        return 0

    lax.fori_loop(0, nch, _chunk, 0)
    plsc.subcore_barrier()

    pltpu.sync_copy(agg_sh.at[pl.ds(s * 640, 640)], zbuf)
    pltpu.sync_copy(zbuf, q_out.at[c, pl.ds(s * 640, 640)])


def _tc_prescale(x, deg2, W1):
    def body(x_ref, deg_ref, w_ref, o_ref):
        dis = lax.rsqrt(deg_ref[0] + deg_ref[1] + 1.0)
        disc = lax.broadcast_in_dim(dis[0:N], (N, D), (0,))
        h = jnp.dot(x_ref[...], w_ref[...], preferred_element_type=_f32)
        o_ref[0:N, :] = h * disc
        o_ref[N:NP, :] = jnp.zeros((NP - N, D), _f32)

    return pl.pallas_call(
        body, out_shape=jax.ShapeDtypeStruct((NP, D), _f32))(x, deg2, W1)


def _tc_mid(p, g1, deg2, b1r, w2r):
    def body(p_ref, g1_ref, deg_ref, b1_ref, w2_ref, o_ref):
        dis = lax.rsqrt(deg_ref[0] + deg_ref[1] + 1.0)
        disc = lax.broadcast_in_dim(dis, (NP, D), (0,))
        z = jnp.maximum(disc * (p_ref[0] + p_ref[1] + g1_ref[...])
                        + b1_ref[...], 0.0)
        h2 = jnp.sum(z * w2_ref[...], axis=1)
        o_ref[...] = dis * h2

    return pl.pallas_call(
        body, out_shape=jax.ShapeDtypeStruct((NP,), _f32))(
            p, g1, deg2, b1r, w2r)


def _tc_final(q, g2, deg2, b2):
    def body(q_ref, g2_ref, deg_ref, b2_ref, o_ref):
        dis = lax.rsqrt(deg_ref[0] + deg_ref[1] + 1.0)
        o_ref[...] = jax.nn.sigmoid(
            dis * (q_ref[0] + q_ref[1] + g2_ref[...]) + b2_ref[...])

    return pl.pallas_call(
        body, out_shape=jax.ShapeDtypeStruct((NP,), _f32))(q, g2, deg2, b2)


def kernel(x, edge_index, W1, b1, W2, b2):
    ei = edge_index.astype(jnp.int32)
    ztail = jnp.zeros((128,), jnp.int32)
    srcE = jnp.concatenate([ei[0], ztail])
    dstE = jnp.concatenate([ei[1], ztail])

    deg2 = _sc_deg(dstE)
    g1 = _tc_prescale(x, deg2, W1)
    p = _sc_agg_rows(srcE, dstE, g1)
    g2 = _tc_mid(p, g1, deg2, b1.reshape(1, D), W2.reshape(1, D))
    q = _sc_agg_scalar(srcE, dstE, g2)
    out = _tc_final(q, g2, deg2, b2)
    return out[:N].reshape(N, 1)


# speedup vs baseline: 95.0874x; 1.0901x over previous
import functools

import jax
import jax.numpy as jnp
from jax import lax
from jax.experimental import pallas as pl
from jax.experimental.pallas import tpu as pltpu
from jax.experimental.pallas import tpu_sc as plsc

N = 10000
NP = 10240
E = 320000
D = 16
EBUF = 10112
NPAIR = 39

_mesh = plsc.VectorSubcoreMesh(core_axis_name="c", subcore_axis_name="s")
_f32 = jnp.float32
_params = pltpu.CompilerParams(use_tc_tiling_on_sc=False,
                               needs_layout_passes=False)


def _worker_slice(w):
    off = jnp.where(w < 4, w * EBUF, 4 * EBUF + (w - 4) * 9984)
    nch = jnp.where(w < 4, 79, 78)
    return off, nch


def _load_edges(ei, row, off, w, buf):
    @pl.when(w < 4)
    def _():
        pltpu.sync_copy(ei.at[row, pl.ds(off, EBUF)], buf)

    @pl.when(w >= 4)
    def _():
        pltpu.sync_copy(ei.at[row, pl.ds(off, 9984)], buf.at[pl.ds(0, 9984)])


@functools.partial(
    pl.kernel,
    out_type=jax.ShapeDtypeStruct((2, NP), _f32),
    mesh=_mesh,
    compiler_params=_params,
    scratch_types=[
        pltpu.VMEM((EBUF,), jnp.int32),
        pltpu.VMEM((128,), _f32),
        pltpu.VMEM((640,), _f32),
        pltpu.VMEM_SHARED((NP,), _f32),
        pltpu.SemaphoreType.DMA,
    ],
)
def _sc_deg(ei, deg_out, idxbuf, ones, zbuf, deg_sh, sem):
    c = lax.axis_index("c")
    s = lax.axis_index("s")
    w = s * 2 + c
    off, nch = _worker_slice(w)

    zero16 = jnp.zeros((16,), _f32)
    one16 = jnp.ones((16,), _f32)

    def _fill_z(i, _):
        zbuf[pl.ds(i * 16, 16)] = zero16
        return 0

    lax.fori_loop(0, 40, _fill_z, 0)

    def _fill_o(i, _):
        ones[pl.ds(i * 16, 16)] = one16
        return 0

    lax.fori_loop(0, 8, _fill_o, 0)

    pltpu.sync_copy(zbuf, deg_sh.at[pl.ds(s * 640, 640)])
    _load_edges(ei, 1, off, w, idxbuf)
    plsc.subcore_barrier()

    def _chunk(j, _):
        dst_idx = idxbuf.at[pl.ds(j * 128, 128)]
        pltpu.async_copy(ones, deg_sh.at[dst_idx], sem, add=True)

        @pl.when(j >= 4)
        def _():
            pltpu.make_async_copy(ones, deg_sh.at[dst_idx], sem).wait()

        return 0

    lax.fori_loop(0, nch, _chunk, 0)
    for _ in range(4):
        pltpu.make_async_copy(ones, deg_sh.at[idxbuf.at[pl.ds(0, 128)]],
                              sem).wait()
    plsc.subcore_barrier()

    pltpu.sync_copy(deg_sh.at[pl.ds(s * 640, 640)], zbuf)
    pltpu.sync_copy(zbuf, deg_out.at[c, pl.ds(s * 640, 640)])


@functools.partial(
    pl.kernel,
    out_type=jax.ShapeDtypeStruct((2, NP, D), _f32),
    mesh=_mesh,
    compiler_params=_params,
    scratch_types=[
        pltpu.VMEM((EBUF,), jnp.int32),
        pltpu.VMEM((EBUF,), jnp.int32),
        pltpu.VMEM((128, D), _f32),
        pltpu.VMEM((128, D), _f32),
        pltpu.VMEM((128, D), _f32),
        pltpu.VMEM((640, D), _f32),
        pltpu.VMEM_SHARED((NP, D), _f32),
        pltpu.VMEM_SHARED((NP, D), _f32),
        pltpu.SemaphoreType.DMA,
        pltpu.SemaphoreType.DMA,
    ],
)
def _sc_agg_rows(ei, g1, p_out, srcbuf, dstbuf, rowsA, rowsB, zbuf,
                 stagebuf, g1_sh, agg_sh, semA, semB):
    c = lax.axis_index("c")
    s = lax.axis_index("s")
    w = s * 2 + c
    off, nch = _worker_slice(w)

    zero16 = jnp.zeros((16,), _f32)

    def _fill_z(i, _):
        zbuf[i, :] = zero16
        return 0

    lax.fori_loop(0, 128, _fill_z, 0)
    for t in range(5):
        pltpu.sync_copy(zbuf, agg_sh.at[pl.ds(s * 640 + t * 128, 128)])

    pltpu.sync_copy(g1.at[pl.ds(s * 640, 640)], stagebuf)
    pltpu.sync_copy(stagebuf, g1_sh.at[pl.ds(s * 640, 640)])

    _load_edges(ei, 0, off, w, srcbuf)
    _load_edges(ei, 1, off, w, dstbuf)
    plsc.subcore_barrier()

    def _src_at(j):
        return srcbuf.at[pl.ds(j * 128, 128)]

    def _dst_at(j):
        return dstbuf.at[pl.ds(j * 128, 128)]

    pltpu.async_copy(g1_sh.at[_src_at(0)], rowsA, semA)

    def _pair(i, carry):
        j0 = 2 * i
        j1 = j0 + 1
        pltpu.make_async_copy(g1_sh.at[_src_at(j0)], rowsA, semA).wait()
        pltpu.async_copy(g1_sh.at[_src_at(j1)], rowsB, semB)
        pltpu.sync_copy(rowsA, agg_sh.at[_dst_at(j0)], add=True)
        pltpu.make_async_copy(g1_sh.at[_src_at(j1)], rowsB, semB).wait()
        pltpu.async_copy(g1_sh.at[_src_at((j1 + 1) % nch)], rowsA, semA)
        pltpu.sync_copy(rowsB, agg_sh.at[_dst_at(j1)], add=True)
        return carry

    lax.fori_loop(0, NPAIR, _pair, 0)
    pltpu.make_async_copy(g1_sh.at[_src_at(0)], rowsA, semA).wait()

    @pl.when(nch == 79)
    def _tail():
        pltpu.sync_copy(rowsA, agg_sh.at[_dst_at(78)], add=True)

    plsc.subcore_barrier()

    for t in range(5):
        pltpu.sync_copy(agg_sh.at[pl.ds(s * 640 + t * 128, 128)], zbuf)
        pltpu.sync_copy(zbuf, p_out.at[c, pl.ds(s * 640 + t * 128, 128)])


@functools.partial(
    pl.kernel,
    out_type=jax.ShapeDtypeStruct((2, NP), _f32),
    mesh=_mesh,
    compiler_params=_params,
    scratch_types=[
        pltpu.VMEM((NP,), _f32),
        pltpu.VMEM((EBUF,), jnp.int32),
        pltpu.VMEM((EBUF,), jnp.int32),
        pltpu.VMEM((128,), _f32),
        pltpu.VMEM((640,), _f32),
        pltpu.VMEM_SHARED((NP,), _f32),
    ],
)
def _sc_agg_scalar(ei, g2, q_out, g2buf, srcbuf, dstbuf, msg, zbuf,
                   agg_sh):
    c = lax.axis_index("c")
    s = lax.axis_index("s")
    w = s * 2 + c
    off, nch = _worker_slice(w)

    zero16 = jnp.zeros((16,), _f32)

    def _fill_z(i, _):
        zbuf[pl.ds(i * 16, 16)] = zero16
        return 0

    lax.fori_loop(0, 40, _fill_z, 0)
    pltpu.sync_copy(zbuf, agg_sh.at[pl.ds(s * 640, 640)])
    pltpu.sync_copy(g2, g2buf)
    _load_edges(ei, 0, off, w, srcbuf)
    _load_edges(ei, 1, off, w, dstbuf)
    plsc.subcore_barrier()

    def _chunk(j, _):
        for g in range(8):
            idx16 = srcbuf[pl.ds(j * 128 + g * 16, 16)]
            msg[pl.ds(g * 16, 16)] = plsc.load_gather(g2buf, [idx16])
        pltpu.sync_copy(msg, agg_sh.at[dstbuf.at[pl.ds(j * 128, 128)]],
                        add=True)
        return 0

    lax.fori_loop(0, nch, _chunk, 0)
    plsc.subcore_barrier()

    pltpu.sync_copy(agg_sh.at[pl.ds(s * 640, 640)], zbuf)
    pltpu.sync_copy(zbuf, q_out.at[c, pl.ds(s * 640, 640)])


def _tc_prescale(x, deg2, W1):
    def body(x_ref, deg_ref, w_ref, o_ref):
        dis = lax.rsqrt(deg_ref[0] + deg_ref[1] + 1.0)
        disc = lax.broadcast_in_dim(dis[0:N], (N, D), (0,))
        h = jnp.dot(x_ref[...], w_ref[...], preferred_element_type=_f32)
        o_ref[0:N, :] = h * disc
        o_ref[N:NP, :] = jnp.zeros((NP - N, D), _f32)

    return pl.pallas_call(
        body, out_shape=jax.ShapeDtypeStruct((NP, D), _f32))(x, deg2, W1)


def _tc_mid(p, g1, deg2, b1r, w2r):
    def body(p_ref, g1_ref, deg_ref, b1_ref, w2_ref, o_ref):
        dis = lax.rsqrt(deg_ref[0] + deg_ref[1] + 1.0)
        disc = lax.broadcast_in_dim(dis, (NP, D), (0,))
        z = jnp.maximum(disc * (p_ref[0] + p_ref[1] + g1_ref[...])
                        + b1_ref[...], 0.0)
        h2 = jnp.sum(z * w2_ref[...], axis=1)
        o_ref[...] = dis * h2

    return pl.pallas_call(
        body, out_shape=jax.ShapeDtypeStruct((NP,), _f32))(
            p, g1, deg2, b1r, w2r)


def _tc_final(q, g2, deg2, b2):
    def body(q_ref, g2_ref, deg_ref, b2_ref, o_ref):
        dis = lax.rsqrt(deg_ref[0] + deg_ref[1] + 1.0)
        o_ref[...] = jax.nn.sigmoid(
            dis * (q_ref[0] + q_ref[1] + g2_ref[...]) + b2_ref[...])

    return pl.pallas_call(
        body, out_shape=jax.ShapeDtypeStruct((NP,), _f32))(q, g2, deg2, b2)


def kernel(x, edge_index, W1, b1, W2, b2):
    ei = edge_index.astype(jnp.int32)

    deg2 = _sc_deg(ei)
    g1 = _tc_prescale(x, deg2, W1)
    p = _sc_agg_rows(ei, g1)
    g2 = _tc_mid(p, g1, deg2, b1.reshape(1, D), W2.reshape(1, D))
    q = _sc_agg_scalar(ei, g2)
    out = _tc_final(q, g2, deg2, b2)
    return out[:N].reshape(N, 1)


# speedup vs baseline: 102.9700x; 1.0829x over previous
import functools

import jax
import jax.numpy as jnp
from jax import lax
from jax.experimental import pallas as pl
from jax.experimental.pallas import tpu as pltpu
from jax.experimental.pallas import tpu_sc as plsc

N = 10000
NP = 10240
E = 320000
D = 16
EBUF = 10112
NPAIR = 39

_mesh = plsc.VectorSubcoreMesh(core_axis_name="c", subcore_axis_name="s")
_f32 = jnp.float32
_params = pltpu.CompilerParams(use_tc_tiling_on_sc=False,
                               needs_layout_passes=False)


def _worker_slice(w):
    off = jnp.where(w < 4, w * EBUF, 4 * EBUF + (w - 4) * 9984)
    nch = jnp.where(w < 4, 79, 78)
    return off, nch


def _load_edges(ei, row, off, w, buf):
    @pl.when(w < 4)
    def _():
        pltpu.sync_copy(ei.at[row, pl.ds(off, EBUF)], buf)

    @pl.when(w >= 4)
    def _():
        pltpu.sync_copy(ei.at[row, pl.ds(off, 9984)], buf.at[pl.ds(0, 9984)])


@functools.partial(
    pl.kernel,
    out_type=jax.ShapeDtypeStruct((2, NP), _f32),
    mesh=_mesh,
    compiler_params=_params,
    scratch_types=[
        pltpu.VMEM((EBUF,), jnp.int32),
        pltpu.VMEM((2528,), _f32),
        pltpu.VMEM((640,), _f32),
        pltpu.VMEM_SHARED((NP,), _f32),
        pltpu.SemaphoreType.DMA,
    ],
)
def _sc_deg(ei, deg_out, idxbuf, ones, zbuf, deg_sh, sem):
    c = lax.axis_index("c")
    s = lax.axis_index("s")
    w = s * 2 + c
    off, nch = _worker_slice(w)

    zero16 = jnp.zeros((16,), _f32)
    one16 = jnp.ones((16,), _f32)

    def _fill_z(i, _):
        zbuf[pl.ds(i * 16, 16)] = zero16
        return 0

    lax.fori_loop(0, 40, _fill_z, 0)

    def _fill_o(i, _):
        ones[pl.ds(i * 16, 16)] = one16
        return 0

    lax.fori_loop(0, 158, _fill_o, 0)

    pltpu.sync_copy(zbuf, deg_sh.at[pl.ds(s * 640, 640)])
    _load_edges(ei, 1, off, w, idxbuf)
    plsc.subcore_barrier()

    def _run(cs):
        for j in range(4):
            pltpu.async_copy(ones.at[pl.ds(0, cs)],
                             deg_sh.at[idxbuf.at[pl.ds(j * cs, cs)]],
                             sem, add=True)
        for j in range(4):
            pltpu.make_async_copy(ones.at[pl.ds(0, cs)],
                                  deg_sh.at[idxbuf.at[pl.ds(j * cs, cs)]],
                                  sem).wait()

    @pl.when(w < 4)
    def _():
        _run(2528)

    @pl.when(w >= 4)
    def _():
        _run(2496)

    plsc.subcore_barrier()

    pltpu.sync_copy(deg_sh.at[pl.ds(s * 640, 640)], zbuf)
    pltpu.sync_copy(zbuf, deg_out.at[c, pl.ds(s * 640, 640)])


@functools.partial(
    pl.kernel,
    out_type=jax.ShapeDtypeStruct((2, NP, D), _f32),
    mesh=_mesh,
    compiler_params=_params,
    scratch_types=[
        pltpu.VMEM((EBUF,), jnp.int32),
        pltpu.VMEM((EBUF,), jnp.int32),
        pltpu.VMEM((1264, D), _f32),
        pltpu.VMEM((1264, D), _f32),
        pltpu.VMEM((640, D), _f32),
        pltpu.VMEM_SHARED((NP, D), _f32),
        pltpu.VMEM_SHARED((NP, D), _f32),
        pltpu.SemaphoreType.DMA,
        pltpu.SemaphoreType.DMA,
    ],
)
def _sc_agg_rows(ei, g1, p_out, srcbuf, dstbuf, rowsA, rowsB,
                 stagebuf, g1_sh, agg_sh, semA, semB):
    c = lax.axis_index("c")
    s = lax.axis_index("s")
    w = s * 2 + c
    off, nch = _worker_slice(w)

    zero16 = jnp.zeros((16,), _f32)

    def _fill_z(i, _):
        stagebuf[i, :] = zero16
        return 0

    lax.fori_loop(0, 640, _fill_z, 0)
    pltpu.sync_copy(stagebuf, agg_sh.at[pl.ds(s * 640, 640)])

    pltpu.sync_copy(g1.at[pl.ds(s * 640, 640)], stagebuf)
    pltpu.sync_copy(stagebuf, g1_sh.at[pl.ds(s * 640, 640)])

    _load_edges(ei, 0, off, w, srcbuf)
    _load_edges(ei, 1, off, w, dstbuf)
    plsc.subcore_barrier()

    def _run(cs):
        bufs = (rowsA, rowsB)
        sems = (semA, semB)

        def _g(j):
            src_idx = srcbuf.at[pl.ds(j * cs, cs)]
            return (g1_sh.at[src_idx], bufs[j % 2].at[pl.ds(0, cs)],
                    sems[j % 2])

        pltpu.async_copy(*_g(0))
        pltpu.async_copy(*_g(1))
        for j in range(8):
            pltpu.make_async_copy(*_g(j)).wait()
            if j + 2 < 8:
                pltpu.async_copy(*_g(j + 2))
            pltpu.sync_copy(bufs[j % 2].at[pl.ds(0, cs)],
                            agg_sh.at[dstbuf.at[pl.ds(j * cs, cs)]],
                            add=True)

    @pl.when(w < 4)
    def _():
        _run(1264)

    @pl.when(w >= 4)
    def _():
        _run(1248)

    plsc.subcore_barrier()

    pltpu.sync_copy(agg_sh.at[pl.ds(s * 640, 640)], stagebuf)
    pltpu.sync_copy(stagebuf, p_out.at[c, pl.ds(s * 640, 640)])


@functools.partial(
    pl.kernel,
    out_type=jax.ShapeDtypeStruct((2, NP), _f32),
    mesh=_mesh,
    compiler_params=_params,
    scratch_types=[
        pltpu.VMEM((NP,), _f32),
        pltpu.VMEM((EBUF,), jnp.int32),
        pltpu.VMEM((EBUF,), jnp.int32),
        pltpu.VMEM((2528,), _f32),
        pltpu.VMEM((640,), _f32),
        pltpu.VMEM_SHARED((NP,), _f32),
    ],
)
def _sc_agg_scalar(ei, g2, q_out, g2buf, srcbuf, dstbuf, msg, zbuf,
                   agg_sh):
    c = lax.axis_index("c")
    s = lax.axis_index("s")
    w = s * 2 + c
    off, nch = _worker_slice(w)

    zero16 = jnp.zeros((16,), _f32)

    def _fill_z(i, _):
        zbuf[pl.ds(i * 16, 16)] = zero16
        return 0

    lax.fori_loop(0, 40, _fill_z, 0)
    pltpu.sync_copy(zbuf, agg_sh.at[pl.ds(s * 640, 640)])
    pltpu.sync_copy(g2, g2buf)
    _load_edges(ei, 0, off, w, srcbuf)
    _load_edges(ei, 1, off, w, dstbuf)
    plsc.subcore_barrier()

    def _run(cs):
        for j in range(4):
            def _fill(g, _):
                idx16 = srcbuf[pl.ds(j * cs + g * 16, 16)]
                msg[pl.ds(g * 16, 16)] = plsc.load_gather(g2buf, [idx16])
                return 0

            lax.fori_loop(0, cs // 16, _fill, 0)
            pltpu.sync_copy(msg.at[pl.ds(0, cs)],
                            agg_sh.at[dstbuf.at[pl.ds(j * cs, cs)]],
                            add=True)

    @pl.when(w < 4)
    def _():
        _run(2528)

    @pl.when(w >= 4)
    def _():
        _run(2496)

    plsc.subcore_barrier()

    pltpu.sync_copy(agg_sh.at[pl.ds(s * 640, 640)], zbuf)
    pltpu.sync_copy(zbuf, q_out.at[c, pl.ds(s * 640, 640)])


def _tc_prescale(x, deg2, W1):
    def body(x_ref, deg_ref, w_ref, o_ref):
        dis = lax.rsqrt(deg_ref[0] + deg_ref[1] + 1.0)
        disc = lax.broadcast_in_dim(dis[0:N], (N, D), (0,))
        h = jnp.dot(x_ref[...], w_ref[...], preferred_element_type=_f32)
        o_ref[0:N, :] = h * disc
        o_ref[N:NP, :] = jnp.zeros((NP - N, D), _f32)

    return pl.pallas_call(
        body, out_shape=jax.ShapeDtypeStruct((NP, D), _f32))(x, deg2, W1)


def _tc_mid(p, g1, deg2, b1r, w2r):
    def body(p_ref, g1_ref, deg_ref, b1_ref, w2_ref, o_ref):
        dis = lax.rsqrt(deg_ref[0] + deg_ref[1] + 1.0)
        disc = lax.broadcast_in_dim(dis, (NP, D), (0,))
        z = jnp.maximum(disc * (p_ref[0] + p_ref[1] + g1_ref[...])
                        + b1_ref[...], 0.0)
        h2 = jnp.sum(z * w2_ref[...], axis=1)
        o_ref[...] = dis * h2

    return pl.pallas_call(
        body, out_shape=jax.ShapeDtypeStruct((NP,), _f32))(
            p, g1, deg2, b1r, w2r)


def _tc_final(q, g2, deg2, b2):
    def body(q_ref, g2_ref, deg_ref, b2_ref, o_ref):
        dis = lax.rsqrt(deg_ref[0] + deg_ref[1] + 1.0)
        o_ref[...] = jax.nn.sigmoid(
            dis * (q_ref[0] + q_ref[1] + g2_ref[...]) + b2_ref[...])

    return pl.pallas_call(
        body, out_shape=jax.ShapeDtypeStruct((NP,), _f32))(q, g2, deg2, b2)


def kernel(x, edge_index, W1, b1, W2, b2):
    ei = edge_index.astype(jnp.int32)

    deg2 = _sc_deg(ei)
    g1 = _tc_prescale(x, deg2, W1)
    p = _sc_agg_rows(ei, g1)
    g2 = _tc_mid(p, g1, deg2, b1.reshape(1, D), W2.reshape(1, D))
    q = _sc_agg_scalar(ei, g2)
    out = _tc_final(q, g2, deg2, b2)
    return out[:N].reshape(N, 1)
